# Initial kernel scaffold; baseline (speedup 1.0000x reference)
import jax
import jax.numpy as jnp
from jax.experimental import pallas as pl


def kernel(v_t, r_gap, s_gap, p_count, C_w):
    raise NotImplementedError("write your pallas kernel here")


# speedup vs baseline: 3.3535x; 3.3535x over previous
import jax
import jax.numpy as jnp
from jax import lax
from jax.experimental import pallas as pl
from jax.experimental.pallas import tpu as pltpu
from jax.experimental.pallas import tpu_sc as plsc

N_CAT = 32
EMB = 64
N_TOTAL = 3 * N_CAT
OUT_D = EMB + N_TOTAL

NC = 2
NS = 16
NW = NC * NS
LANES = 16

CH = 256


def _sc_body(v_hbm, r_hbm, s_hbm, p_hbm, t_hbm, out_hbm,
             T_v, r_v, s_v, p_v, v_v, o_v):
    cid = lax.axis_index("c")
    sid = lax.axis_index("s")
    wid = sid * NC + cid
    tok_per_w = r_hbm.shape[0] // NW
    n_chunks = tok_per_w // CH
    base0 = wid * tok_per_w

    pltpu.sync_copy(t_hbm, T_v)

    zeros = jnp.zeros((LANES,), jnp.float32)
    ones = jnp.ones((LANES,), jnp.float32)

    @plsc.parallel_loop(0, CH, unroll=4)
    def _(t):
        for k in range(6):
            o_v[pl.ds(t * OUT_D + EMB + k * LANES, LANES)] = zeros

    @pl.loop(0, n_chunks)
    def _(c):
        base = base0 + c * CH
        pltpu.sync_copy(r_hbm.at[pl.ds(base, CH)], r_v)
        pltpu.sync_copy(s_hbm.at[pl.ds(base, CH)], s_v)
        pltpu.sync_copy(p_hbm.at[pl.ds(base, CH)], p_v)
        pltpu.sync_copy(v_hbm.at[pl.ds(base * EMB, CH * EMB)], v_v)

        @plsc.parallel_loop(0, CH // LANES)
        def _(g):
            t0 = g * LANES
            toks = lax.iota(jnp.int32, LANES) + t0
            r16 = r_v[pl.ds(t0, LANES)]
            s16 = s_v[pl.ds(t0, LANES)] + N_CAT
            p16 = p_v[pl.ds(t0, LANES)] + 2 * N_CAT
            for j in range(LANES):
                t = t0 + j
                rb = r16[j] * EMB
                sb = s16[j] * EMB
                pb = p16[j] * EMB
                for k in range(EMB // LANES):
                    o = k * LANES
                    acc = T_v[pl.ds(rb + o, LANES)] \
                        + T_v[pl.ds(sb + o, LANES)] \
                        + T_v[pl.ds(pb + o, LANES)]
                    o_v[pl.ds(t * OUT_D + o, LANES)] = \
                        v_v[pl.ds(t * EMB + o, LANES)] * acc
            pos = toks * OUT_D + EMB
            plsc.store_scatter(o_v, [pos + r16], ones)
            plsc.store_scatter(o_v, [pos + s16], ones)
            plsc.store_scatter(o_v, [pos + p16], ones)

        pltpu.sync_copy(o_v, out_hbm.at[pl.ds(base * OUT_D, CH * OUT_D)])

        @plsc.parallel_loop(0, CH // LANES, unroll=2)
        def _(g):
            t0 = g * LANES
            toks = lax.iota(jnp.int32, LANES) + t0
            pos = toks * OUT_D + EMB
            plsc.store_scatter(o_v, [pos + r_v[pl.ds(t0, LANES)]], zeros)
            plsc.store_scatter(o_v, [pos + s_v[pl.ds(t0, LANES)] + N_CAT],
                               zeros)
            plsc.store_scatter(o_v, [pos + p_v[pl.ds(t0, LANES)] + 2 * N_CAT],
                               zeros)


@jax.jit
def _run(v2, r, s, p, T):
    bl = r.shape[0]
    mesh = plsc.VectorSubcoreMesh(core_axis_name="c", subcore_axis_name="s",
                                  num_cores=NC, num_subcores=NS)
    f = pl.kernel(
        _sc_body,
        out_type=jax.ShapeDtypeStruct((bl * OUT_D,), jnp.float32),
        mesh=mesh,
        compiler_params=pltpu.CompilerParams(needs_layout_passes=False),
        scratch_types=[
            pltpu.VMEM((N_TOTAL * EMB,), jnp.float32),
            pltpu.VMEM((CH,), jnp.int32),
            pltpu.VMEM((CH,), jnp.int32),
            pltpu.VMEM((CH,), jnp.int32),
            pltpu.VMEM((CH * EMB,), jnp.float32),
            pltpu.VMEM((CH * OUT_D,), jnp.float32),
        ],
    )
    return f(v2, r, s, p, T)


def kernel(v_t, r_gap, s_gap, p_count, C_w):
    B, L, E = v_t.shape
    bl = B * L
    v2 = v_t.reshape(bl * E)
    r = r_gap.reshape(bl).astype(jnp.int32)
    s = s_gap.reshape(bl).astype(jnp.int32)
    p = p_count.reshape(bl).astype(jnp.int32)
    T = C_w.T.reshape(N_TOTAL * EMB).astype(jnp.float32)
    out = _run(v2, r, s, p, T)
    return out.reshape(B, L, OUT_D)
